# Initial kernel scaffold; baseline (speedup 1.0000x reference)
import jax
import jax.numpy as jnp
from jax.experimental import pallas as pl


def kernel(x, emb, W1, b1, W2, b2):
    raise NotImplementedError("write your pallas kernel here")


# speedup vs baseline: 1.2629x; 1.2629x over previous
import functools

import jax
import jax.numpy as jnp
from jax import lax
from jax.experimental import pallas as pl
from jax.experimental.pallas import tpu as pltpu
from jax.experimental.pallas import tpu_sc as plsc

NUM_CHARACTERS = 1000000
EMBED_DIM = 16
BATCH = 16384
NUM_STAGES = 64

_NC = 2
_NS = 16
_NW = _NC * _NS
_B2 = 2 * BATCH
_BPW = _B2 // _NW
_CHUNK = 128
_NCHUNK = _BPW // _CHUNK


@functools.partial(
    pl.kernel,
    out_type=jax.ShapeDtypeStruct((_B2, EMBED_DIM), jnp.float32),
    mesh=plsc.VectorSubcoreMesh(
        core_axis_name="c", subcore_axis_name="s",
        num_cores=_NC, num_subcores=_NS),
    scratch_types=[
        pltpu.VMEM((_BPW,), jnp.int32),
        pltpu.VMEM((_BPW, EMBED_DIM), jnp.float32),
        pltpu.SemaphoreType.DMA,
    ],
    compiler_params=pltpu.CompilerParams(use_tc_tiling_on_sc=False),
)
def _sc_gather(idx_hbm, table_hbm, out_hbm, idx_v, rows_v, sem):
    wid = lax.axis_index("s") * _NC + lax.axis_index("c")
    base = wid * _BPW
    pltpu.sync_copy(idx_hbm.at[pl.ds(base, _BPW)], idx_v)
    copies = []
    for j in range(_NCHUNK):
        sl = pl.ds(j * _CHUNK, _CHUNK)
        copies.append(
            pltpu.async_copy(table_hbm.at[idx_v.at[sl]], rows_v.at[sl], sem))
    for c in copies:
        c.wait()
    pltpu.sync_copy(rows_v, out_hbm.at[pl.ds(base, _BPW)])


def _mlp_body(h_ref, w1_ref, b1_ref, w2_ref, b2_ref, o_ref):
    z = jnp.dot(h_ref[...], w1_ref[...], preferred_element_type=jnp.float32)
    z = jnp.maximum(z + b1_ref[...], 0.0)
    o_ref[...] = (
        jnp.dot(z, w2_ref[...], preferred_element_type=jnp.float32)
        + b2_ref[...])


_MLP_BLK = 2048


def _mlp(h, W1, b1, W2, b2):
    return pl.pallas_call(
        _mlp_body,
        grid=(BATCH // _MLP_BLK,),
        in_specs=[
            pl.BlockSpec((_MLP_BLK, 2 * EMBED_DIM), lambda i: (i, 0)),
            pl.BlockSpec((2 * EMBED_DIM, 64), lambda i: (0, 0)),
            pl.BlockSpec((1, 64), lambda i: (0, 0)),
            pl.BlockSpec((64, NUM_STAGES), lambda i: (0, 0)),
            pl.BlockSpec((1, NUM_STAGES), lambda i: (0, 0)),
        ],
        out_specs=pl.BlockSpec((_MLP_BLK, NUM_STAGES), lambda i: (i, 0)),
        out_shape=jax.ShapeDtypeStruct((BATCH, NUM_STAGES), jnp.float32),
    )(h, W1, b1.reshape(1, 64), W2, b2.reshape(1, NUM_STAGES))


def kernel(x, emb, W1, b1, W2, b2):
    idx = x.reshape(-1).astype(jnp.int32)
    rows = _sc_gather(idx, emb)
    h = rows.reshape(BATCH, 2 * EMBED_DIM)
    return _mlp(h, W1, b1, W2, b2)
